# baseline (speedup 1.0000x reference)
import functools

import jax
import jax.numpy as jnp
from jax import lax
from jax.experimental import pallas as pl
from jax.experimental.pallas import tpu as pltpu
from jax.experimental.pallas import tpu_sc as plsc

B = 16384
NUM_FEAT = 32
CAT_VOCAB = 1001
CAT_DIM = 16
EMB_OTHER = 64
HIDDEN = 128
PACK = 128


def _sc_gather(xc0, xc1, nidc, nidm, pcd_t, ct_t, card_p, merch_p):
    info = plsc.get_sparse_core_info()
    NC, NS = info.num_cores, info.num_subcores
    NW = NC * NS
    n = B // NW
    CH = 128
    nchunk = n // CH

    mesh = plsc.VectorSubcoreMesh(core_axis_name="c", subcore_axis_name="s")

    @functools.partial(
        pl.kernel,
        mesh=mesh,
        compiler_params=pltpu.CompilerParams(needs_layout_passes=False),
        out_type=[
            jax.ShapeDtypeStruct((CAT_DIM, B), jnp.float32),
            jax.ShapeDtypeStruct((CAT_DIM, B), jnp.float32),
            jax.ShapeDtypeStruct((B, PACK), jnp.int32),
            jax.ShapeDtypeStruct((B, PACK), jnp.int32),
        ],
        scratch_types=[
            pltpu.VMEM((4, B // (2 * 16)), jnp.int32),
            pltpu.VMEM((CAT_DIM, CAT_VOCAB), jnp.float32),
            pltpu.VMEM((CAT_DIM, CAT_VOCAB), jnp.float32),
            pltpu.VMEM((CAT_DIM, B // (2 * 16)), jnp.float32),
            pltpu.VMEM((CAT_DIM, B // (2 * 16)), jnp.float32),
            pltpu.VMEM((2, 128, PACK), jnp.int32),
            pltpu.VMEM((2, 128, PACK), jnp.int32),
            pltpu.SemaphoreType.DMA,
        ],
    )
    def k(xc0_h, xc1_h, nidc_h, nidm_h, pcd_h, ct_h, card_h, merch_h,
          e0_o, e1_o, card_o, merch_o,
          idx_v, pcd_v, ct_v, e0_b, e1_b, card_b, merch_b, sem):
        wid = lax.axis_index("s") * NC + lax.axis_index("c")
        base = wid * n
        src = pl.ds(base, n)
        pltpu.sync_copy(xc0_h.at[src], idx_v.at[0])
        pltpu.sync_copy(xc1_h.at[src], idx_v.at[1])
        pltpu.sync_copy(nidc_h.at[src], idx_v.at[2])
        pltpu.sync_copy(nidm_h.at[src], idx_v.at[3])
        pltpu.sync_copy(pcd_h, pcd_v)
        pltpu.sync_copy(ct_h, ct_v)

        def drain_and_flush(c):
            s = c % 2
            pltpu.make_async_copy(card_h.at[pl.ds(0, CH), :], card_b.at[s], sem).wait()
            pltpu.make_async_copy(merch_h.at[pl.ds(0, CH), :], merch_b.at[s], sem).wait()
            out = pl.ds(base + c * CH, CH)
            pltpu.sync_copy(card_b.at[s], card_o.at[out])
            pltpu.sync_copy(merch_b.at[s], merch_o.at[out])

        for c in range(nchunk):
            s = c % 2

            def issue(g, _):
                qb = c * CH + g * 16
                vc = lax.shift_right_logical(idx_v[2, pl.ds(qb, 16)], 2)
                vm = lax.shift_right_logical(idx_v[3, pl.ds(qb, 16)], 2)
                for lane in range(16):
                    row = pl.ds(g * 16 + lane, 1)
                    pltpu.async_copy(card_h.at[pl.ds(vc[lane], 1), :], card_b.at[s, row, :], sem)
                    pltpu.async_copy(merch_h.at[pl.ds(vm[lane], 1), :], merch_b.at[s, row, :], sem)
                return _

            lax.fori_loop(0, CH // 16, issue, None)
            if c > 0:
                drain_and_flush(c - 1)

        def egroup(g, _):
            qb = g * 16
            v0 = jnp.clip(idx_v[0, pl.ds(qb, 16)] + 1, 0, CAT_VOCAB - 1)
            v1 = jnp.clip(idx_v[1, pl.ds(qb, 16)] + 1, 0, CAT_VOCAB - 1)
            for kk in range(CAT_DIM):
                kv = jnp.full((16,), kk, jnp.int32)
                e0_b[kk, pl.ds(qb, 16)] = plsc.load_gather(pcd_v, [kv, v0])
                e1_b[kk, pl.ds(qb, 16)] = plsc.load_gather(ct_v, [kv, v1])
            return _

        lax.fori_loop(0, n // 16, egroup, None)
        drain_and_flush(nchunk - 1)
        out = pl.ds(base, n)
        pltpu.sync_copy(e0_b, e0_o.at[:, out])
        pltpu.sync_copy(e1_b, e1_o.at[:, out])

    return k(xc0, xc1, nidc, nidm, pcd_t, ct_t, card_p, merch_p)


_BLK = 1024


def _dgt(a, w):
    return lax.dot_general(a, w, dimension_numbers=(((0,), (0,)), ((), ())),
                           preferred_element_type=jnp.float32)


def _unpack_dot(packed_ref, idx_ref, w_even, w_odd):
    q = idx_ref[:] & 3
    half = jnp.where(q >= 2, packed_ref[:, EMB_OTHER:], packed_ref[:, :EMB_OTHER])
    quart = jnp.where((q & 1) == 1, half[:, EMB_OTHER // 2:], half[:, :EMB_OTHER // 2])
    even = lax.bitcast_convert_type(lax.shift_left(quart, 16), jnp.float32)
    odd = lax.bitcast_convert_type(quart & jnp.int32(-65536), jnp.float32)
    return (jnp.dot(even, w_even[:], preferred_element_type=jnp.float32)
            + jnp.dot(odd, w_odd[:], preferred_element_type=jnp.float32))


def _tc_body(xn, e0r, e1r, cr, mr, pc, pm, wt, bt, wce, wco, bc, wme, wmo, bm,
             to, co, mo):
    acc = _dgt(xn[:], wt[pl.ds(0, NUM_FEAT), :])
    acc += _dgt(e0r[:], wt[pl.ds(NUM_FEAT, CAT_DIM), :])
    acc += _dgt(e1r[:], wt[pl.ds(NUM_FEAT + CAT_DIM, CAT_DIM), :])
    to[:] = acc + bt[:]
    co[:] = _unpack_dot(cr, pc, wce, wco) + bc[:]
    mo[:] = _unpack_dot(mr, pm, wme, wmo) + bm[:]


def _tc_forward(x_num_t, e0_t, e1_t, card_p, merch_p, nidc_col, nidm_col,
                W_trans, b_trans, wce, wco, b_card, wme, wmo, b_merch):
    grid = (B // _BLK,)
    col_blk = lambda h: pl.BlockSpec((h, _BLK), lambda i: (0, i))
    row_blk = lambda w: pl.BlockSpec((_BLK, w), lambda i: (i, 0))
    full = lambda a: pl.BlockSpec(a.shape, lambda i: (0,) * a.ndim)
    return pl.pallas_call(
        _tc_body,
        grid=grid,
        in_specs=[
            col_blk(NUM_FEAT), col_blk(CAT_DIM), col_blk(CAT_DIM),
            row_blk(PACK), row_blk(PACK), row_blk(1), row_blk(1),
            full(W_trans), full(b_trans), full(wce), full(wco), full(b_card),
            full(wme), full(wmo), full(b_merch),
        ],
        out_specs=[row_blk(HIDDEN), row_blk(HIDDEN), row_blk(HIDDEN)],
        out_shape=[jax.ShapeDtypeStruct((B, HIDDEN), jnp.float32)] * 3,
    )(x_num_t, e0_t, e1_t, card_p, merch_p, nidc_col, nidm_col,
      W_trans, b_trans, wce, wco, b_card, wme, wmo, b_merch)


def kernel(x_num, x_cat, n_id_card, n_id_merchant,
           emb_pcd, emb_ct, W_trans, b_trans,
           emb_card, W_card, b_card,
           emb_merch, W_merch, b_merch):
    xc0 = x_cat[:, 0].astype(jnp.int32)
    xc1 = x_cat[:, 1].astype(jnp.int32)
    nidc = n_id_card.astype(jnp.int32)
    nidm = n_id_merchant.astype(jnp.int32)
    pack = lambda t: lax.bitcast_convert_type(
        t.astype(jnp.bfloat16).reshape(-1, EMB_OTHER // 2, 2),
        jnp.int32).reshape(-1, PACK)
    card_pk = pack(emb_card)
    merch_pk = pack(emb_merch)
    e0_t, e1_t, card_p, merch_p = _sc_gather(
        xc0, xc1, nidc, nidm, emb_pcd.T, emb_ct.T, card_pk, merch_pk)
    b_t = b_trans.reshape(1, HIDDEN)
    b_c = b_card.reshape(1, HIDDEN)
    b_m = b_merch.reshape(1, HIDDEN)
    return _tc_forward(x_num.T, e0_t, e1_t, card_p, merch_p,
                       nidc.reshape(B, 1), nidm.reshape(B, 1),
                       W_trans, b_t, W_card[0::2, :], W_card[1::2, :], b_c,
                       W_merch[0::2, :], W_merch[1::2, :], b_m)


# speedup vs baseline: 4.4783x; 4.4783x over previous
import functools

import jax
import jax.numpy as jnp
from jax import lax
from jax.experimental import pallas as pl
from jax.experimental.pallas import tpu as pltpu
from jax.experimental.pallas import tpu_sc as plsc

B = 16384
NUM_FEAT = 32
CAT_VOCAB = 1001
CAT_DIM = 16
EMB_OTHER = 64
HIDDEN = 128

_IDX_CHUNK = 128


def _sc_gather(xc0, xc1, nidc, nidm, emb_pcd, emb_ct, emb_card, emb_merch):
    info = plsc.get_sparse_core_info()
    NC, NS = info.num_cores, info.num_subcores
    NW = NC * NS
    n = B // NW
    CH = 64
    nchunk = n // CH

    mesh = plsc.VectorSubcoreMesh(core_axis_name="c", subcore_axis_name="s")

    @functools.partial(
        pl.kernel,
        mesh=mesh,
        out_type=[
            jax.ShapeDtypeStruct((B, CAT_DIM), jnp.float32),
            jax.ShapeDtypeStruct((B, CAT_DIM), jnp.float32),
            jax.ShapeDtypeStruct((B, EMB_OTHER), jnp.float32),
            jax.ShapeDtypeStruct((B, EMB_OTHER), jnp.float32),
        ],
        scratch_types=[
            pltpu.VMEM((4, B // (2 * 16)), jnp.int32),
            pltpu.VMEM((2, CH, CAT_DIM), jnp.float32),
            pltpu.VMEM((2, CH, CAT_DIM), jnp.float32),
            pltpu.VMEM((2, CH, EMB_OTHER), jnp.float32),
            pltpu.VMEM((2, CH, EMB_OTHER), jnp.float32),
            pltpu.SemaphoreType.DMA,
        ],
    )
    def k(xc0_h, xc1_h, nidc_h, nidm_h, pcd_h, ct_h, card_h, merch_h,
          e0_o, e1_o, card_o, merch_o,
          idx_v, e0_b, e1_b, card_b, merch_b, sem):
        wid = lax.axis_index("s") * NC + lax.axis_index("c")
        base = wid * n
        src = pl.ds(base, n)
        pltpu.sync_copy(xc0_h.at[src], idx_v.at[0])
        pltpu.sync_copy(xc1_h.at[src], idx_v.at[1])
        pltpu.sync_copy(nidc_h.at[src], idx_v.at[2])
        pltpu.sync_copy(nidm_h.at[src], idx_v.at[3])

        def drain_and_flush(c):
            s = c % 2
            pltpu.make_async_copy(pcd_h.at[pl.ds(0, CH), :], e0_b.at[s], sem).wait()
            pltpu.make_async_copy(ct_h.at[pl.ds(0, CH), :], e1_b.at[s], sem).wait()
            pltpu.make_async_copy(card_h.at[pl.ds(0, CH), :], card_b.at[s], sem).wait()
            pltpu.make_async_copy(merch_h.at[pl.ds(0, CH), :], merch_b.at[s], sem).wait()
            out = pl.ds(base + c * CH, CH)
            pltpu.sync_copy(e0_b.at[s], e0_o.at[out])
            pltpu.sync_copy(e1_b.at[s], e1_o.at[out])
            pltpu.sync_copy(card_b.at[s], card_o.at[out])
            pltpu.sync_copy(merch_b.at[s], merch_o.at[out])

        for c in range(nchunk):
            s = c % 2

            def issue_group(g, _):
                qb = c * CH + g * 16
                v0 = jnp.clip(idx_v[0, pl.ds(qb, 16)] + 1, 0, CAT_VOCAB - 1)
                v1 = jnp.clip(idx_v[1, pl.ds(qb, 16)] + 1, 0, CAT_VOCAB - 1)
                vc = idx_v[2, pl.ds(qb, 16)]
                vm = idx_v[3, pl.ds(qb, 16)]
                for lane in range(16):
                    row = pl.ds(g * 16 + lane, 1)
                    pltpu.async_copy(pcd_h.at[pl.ds(v0[lane], 1), :], e0_b.at[s, row, :], sem)
                    pltpu.async_copy(ct_h.at[pl.ds(v1[lane], 1), :], e1_b.at[s, row, :], sem)
                    pltpu.async_copy(card_h.at[pl.ds(vc[lane], 1), :], card_b.at[s, row, :], sem)
                    pltpu.async_copy(merch_h.at[pl.ds(vm[lane], 1), :], merch_b.at[s, row, :], sem)
                return _

            lax.fori_loop(0, CH // 16, issue_group, None)
            if c > 0:
                drain_and_flush(c - 1)
        drain_and_flush(nchunk - 1)

    return k(xc0, xc1, nidc, nidm, emb_pcd, emb_ct, emb_card, emb_merch)


_BLK = 1024


def _tc_body(xn, e0r, e1r, cr, mr, wt, bt, wc, bc, wm, bm, to, co, mo):
    acc = jnp.dot(xn[:], wt[pl.ds(0, NUM_FEAT), :],
                  preferred_element_type=jnp.float32)
    acc += jnp.dot(e0r[:], wt[pl.ds(NUM_FEAT, CAT_DIM), :],
                   preferred_element_type=jnp.float32)
    acc += jnp.dot(e1r[:], wt[pl.ds(NUM_FEAT + CAT_DIM, CAT_DIM), :],
                   preferred_element_type=jnp.float32)
    to[:] = acc + bt[:]
    co[:] = jnp.dot(cr[:], wc[:], preferred_element_type=jnp.float32) + bc[:]
    mo[:] = jnp.dot(mr[:], wm[:], preferred_element_type=jnp.float32) + bm[:]


def _tc_forward(x_num, e0, e1, card_rows, merch_rows,
                W_trans, b_trans, W_card, b_card, W_merch, b_merch):
    grid = (B // _BLK,)
    row_blk = lambda w: pl.BlockSpec((_BLK, w), lambda i: (i, 0))
    full = lambda a: pl.BlockSpec(a.shape, lambda i: (0,) * a.ndim)
    return pl.pallas_call(
        _tc_body,
        grid=grid,
        in_specs=[
            row_blk(NUM_FEAT), row_blk(CAT_DIM), row_blk(CAT_DIM),
            row_blk(EMB_OTHER), row_blk(EMB_OTHER),
            full(W_trans), full(b_trans), full(W_card), full(b_card),
            full(W_merch), full(b_merch),
        ],
        out_specs=[row_blk(HIDDEN), row_blk(HIDDEN), row_blk(HIDDEN)],
        out_shape=[jax.ShapeDtypeStruct((B, HIDDEN), jnp.float32)] * 3,
    )(x_num, e0, e1, card_rows, merch_rows,
      W_trans, b_trans, W_card, b_card, W_merch, b_merch)


def kernel(x_num, x_cat, n_id_card, n_id_merchant,
           emb_pcd, emb_ct, W_trans, b_trans,
           emb_card, W_card, b_card,
           emb_merch, W_merch, b_merch):
    xc0 = x_cat[:, 0].astype(jnp.int32)
    xc1 = x_cat[:, 1].astype(jnp.int32)
    e0, e1, card_rows, merch_rows = _sc_gather(
        xc0, xc1, n_id_card.astype(jnp.int32), n_id_merchant.astype(jnp.int32),
        emb_pcd, emb_ct, emb_card, emb_merch)
    b_t = b_trans.reshape(1, HIDDEN)
    b_c = b_card.reshape(1, HIDDEN)
    b_m = b_merch.reshape(1, HIDDEN)
    return _tc_forward(x_num, e0, e1, card_rows, merch_rows,
                       W_trans, b_t, W_card, b_c, W_merch, b_m)
